# Initial kernel scaffold; baseline (speedup 1.0000x reference)
import jax
import jax.numpy as jnp
from jax.experimental import pallas as pl


def kernel(x, elem, W1, b1, W2, b2, Wa, ba, idx):
    raise NotImplementedError("write your pallas kernel here")


# speedup vs baseline: 3.5790x; 3.5790x over previous
import functools

import jax
import jax.numpy as jnp
from jax import lax
from jax.experimental import pallas as pl
from jax.experimental.pallas import tpu as pltpu
from jax.experimental.pallas import tpu_sc as plsc

_N = 10000
_ROW = 128
_NACC = 10112
_HALF = 5056
_FROWS = 5120
_ACCR = 5248
_RSROWS = 88
_CHUNK = 128
_NCHUNK = 79
_EPT = _CHUNK * _NCHUNK
_EPAD = 32 * _EPT
_ZROWS = _ACCR // 16


def _dense_body(x_ref, w1_ref, b1_ref, w2_ref, b2_ref, wa_ref, f_ref, a_ref, m_ref):
    h = jnp.maximum(
        jnp.dot(x_ref[...], w1_ref[...], preferred_element_type=jnp.float32)
        + b1_ref[...], 0.0)
    f = jnp.dot(h, w2_ref[...], preferred_element_type=jnp.float32) + b2_ref[...]
    f_ref[...] = f
    a = jnp.dot(f, wa_ref[...], preferred_element_type=jnp.float32)
    a_ref[...] = a
    m_ref[...] = jnp.broadcast_to(jnp.max(a, axis=0, keepdims=True), (8, 128))[None]


def _dense(x, w1c, b1c, w2bd, b2c, wad):
    return pl.pallas_call(
        _dense_body,
        grid=(10,),
        in_specs=[
            pl.BlockSpec((1000, 128), lambda i: (i, 0)),
            pl.BlockSpec((128, 256), lambda i: (0, 0)),
            pl.BlockSpec((1, 256), lambda i: (0, 0)),
            pl.BlockSpec((256, 128), lambda i: (0, 0)),
            pl.BlockSpec((1, 128), lambda i: (0, 0)),
            pl.BlockSpec((128, 128), lambda i: (0, 0)),
        ],
        out_specs=[
            pl.BlockSpec((1000, 128), lambda i: (i, 0)),
            pl.BlockSpec((1000, 128), lambda i: (i, 0)),
            pl.BlockSpec((1, 8, 128), lambda i: (i, 0, 0)),
        ],
        out_shape=[
            jax.ShapeDtypeStruct((_N, 128), jnp.float32),
            jax.ShapeDtypeStruct((_N, 128), jnp.float32),
            jax.ShapeDtypeStruct((10, 8, 128), jnp.float32),
        ],
    )(x, w1c, b1c, w2bd, b2c, wad)


def _sc_edge_body(htab, g0h, g1h, sdp, elemp, params,
                  fout,
                  sd_v, src_v, dst_v, elem_v, prm_v,
                  rows_v, zbuf_v, g0_v, g1_v, rs_v, idxa_v, acc_sh, sem):
    cid = lax.axis_index("c")
    sid = lax.axis_index("s")
    wid = sid * 2 + cid

    zero16 = jnp.zeros((16,), jnp.float32)
    lane = lax.iota(jnp.int32, 16)
    r0 = sid * _ZROWS

    def _zrow(j, carry):
        for k in range(_ROW // 16):
            zbuf_v[j, pl.ds(k * 16, 16)] = zero16
        return carry

    lax.fori_loop(0, _CHUNK, _zrow, 0)

    def _zero_acc_slice():
        for off, size in ((0, 128), (128, 128), (256, 72)):
            pltpu.sync_copy(zbuf_v.at[pl.ds(0, size)], acc_sh.at[pl.ds(r0 + off, size)])

    def _zero_rs():
        def _zrs(j, carry):
            for k in range(8):
                rs_v[j, pl.ds(k * 16, 16)] = zero16
            return carry

        lax.fori_loop(0, _RSROWS, _zrs, 0)

    _zero_acc_slice()
    _zero_rs()

    for gq in range(5):
        idxa_v[pl.ds(gq * 16, 16)] = _FROWS + gq * 16 + lane

    pltpu.sync_copy(g0h, g0_v)
    pltpu.sync_copy(g1h, g1_v)
    pltpu.sync_copy(params, prm_v)
    pv = prm_v[...]
    c0 = jnp.full((16,), pv[0])
    c1 = jnp.full((16,), pv[1])

    plsc.subcore_barrier()

    ebase = wid * _EPT

    for ph in range(2):
        nbase = ph * _HALF

        def _chunk(t, carry):
            base = ebase + t * _CHUNK
            pltpu.sync_copy(sdp.at[pl.ds(base, _CHUNK)], sd_v)
            pltpu.sync_copy(elemp.at[pl.ds(base, _CHUNK)], elem_v)
            for gq in range(_CHUNK // 16):
                v = sd_v[pl.ds(gq * 16, 16)]
                sl = (v >> 16) - nbase
                ok = (sl >= 0) & (sl < _HALF)
                src_v[pl.ds(gq * 16, 16)] = jnp.where(ok, sl, _HALF)
                dst_v[pl.ds(gq * 16, 16)] = v & 0xFFFF
            pltpu.async_copy(htab.at[dst_v], rows_v, sem).wait()

            def _grp(gq, c2):
                ev = elem_v[pl.ds(gq * 16, 16)]
                e0g = jnp.exp(c0 * ev)
                e1g = jnp.exp(c1 * ev)
                sv = src_v[pl.ds(gq * 16, 16)]
                dv = dst_v[pl.ds(gq * 16, 16)]
                for l in range(16):
                    j = gq * 16 + l
                    d = dv[l]
                    s = sv[l]
                    w0s = g0_v[pl.ds(d, 16)][0] * e0g[l]
                    w1s = g1_v[pl.ds(d, 16)][0] * e1g[l]
                    w0 = jnp.full((16,), w0s)
                    w1 = jnp.full((16,), w1s)
                    for k in range(_ROW // 16):
                        scale = w0 if k < 4 else w1
                        rows_v[j, pl.ds(k * 16, 16)] = rows_v[j, pl.ds(k * 16, 16)] * scale
                    p2 = 2 * s
                    r = p2 // 128
                    cb = p2 % 128
                    cb2 = jnp.minimum(cb, 112)
                    rel = jnp.full((16,), cb - cb2)
                    upd = (jnp.where(lane == rel, w0, 0.0)
                           + jnp.where(lane == rel + 1, w1, 0.0))
                    rs_v[r, pl.ds(cb2, 16)] = rs_v[r, pl.ds(cb2, 16)] + upd
                return c2

            lax.fori_loop(0, _CHUNK // 16, _grp, 0)
            pltpu.sync_copy(rows_v, acc_sh.at[src_v], add=True)
            return carry

        lax.fori_loop(0, _NCHUNK, _chunk, 0)

        pltpu.sync_copy(rs_v.at[pl.ds(0, 80)], acc_sh.at[idxa_v], add=True)
        plsc.subcore_barrier()
        pltpu.sync_copy(acc_sh.at[pl.ds(r0, _ZROWS)], fout.at[cid, ph, pl.ds(r0, _ZROWS)])
        if ph == 0:
            _zero_acc_slice()
            _zero_rs()
            plsc.subcore_barrier()


_sc_edge = pl.kernel(
    _sc_edge_body,
    out_type=jax.ShapeDtypeStruct((2, 2, _ACCR, _ROW), jnp.float32),
    mesh=plsc.VectorSubcoreMesh(core_axis_name="c", subcore_axis_name="s"),
    scratch_types=[
        pltpu.VMEM((_CHUNK,), jnp.int32),
        pltpu.VMEM((_CHUNK,), jnp.int32),
        pltpu.VMEM((_CHUNK,), jnp.int32),
        pltpu.VMEM((_CHUNK,), jnp.float32),
        pltpu.VMEM((16,), jnp.float32),
        pltpu.VMEM((_CHUNK, _ROW), jnp.float32),
        pltpu.VMEM((_CHUNK, _ROW), jnp.float32),
        pltpu.VMEM((_NACC + 16,), jnp.float32),
        pltpu.VMEM((_NACC + 16,), jnp.float32),
        pltpu.VMEM((_RSROWS, 128), jnp.float32),
        pltpu.VMEM((80,), jnp.int32),
        pltpu.VMEM_SHARED((_ACCR, _ROW), jnp.float32),
        pltpu.SemaphoreType.DMA,
    ],
)


def kernel(x, elem, W1, b1, W2, b2, Wa, ba, idx):
    f32 = jnp.float32
    w1c = jnp.concatenate([W1[0], W1[1]], axis=1)
    b1c = jnp.concatenate([b1[0], b1[1]])[None, :]
    w2bd = jnp.zeros((256, 128), f32)
    w2bd = w2bd.at[0:128, 0:64].set(W2[0]).at[128:256, 64:128].set(W2[1])
    b2c = jnp.concatenate([b2[0], b2[1]])[None, :]
    wad = jnp.zeros((128, 128), f32)
    wad = wad.at[0:64, 0].set(Wa[0, 64:128, 0]).at[64:128, 1].set(Wa[1, 64:128, 0])

    feats, araw, amax_blk = _dense(x, w1c, b1c, w2bd, b2c, wad)
    a_max = jnp.max(amax_blk[:, 0, :2], axis=0)
    g = jnp.exp(araw[:, :2] - a_max[None, :])

    htab = jnp.zeros((_NACC, _ROW), f32)
    htab = htab.at[:_N, :].set(feats)
    g0h = jnp.zeros((_NACC + 16,), f32).at[:_N].set(g[:, 0])
    g1h = jnp.zeros((_NACC + 16,), f32).at[:_N].set(g[:, 1])

    params = jnp.zeros((16,), f32).at[0].set(Wa[0, 128, 0]).at[1].set(Wa[1, 128, 0])

    npad = _EPAD - idx.shape[1]
    srcp = jnp.concatenate([idx[0], jnp.full((npad,), _N, jnp.int32)])
    dstp = jnp.concatenate([idx[1], jnp.full((npad,), _N, jnp.int32)])
    sdp = (srcp << 16) | dstp
    elemp = jnp.concatenate([elem, jnp.zeros((npad,), f32)])

    fpart = _sc_edge(htab, g0h, g1h, sdp, elemp, params)
    p = fpart[0] + fpart[1]
    feats_acc = jnp.concatenate([p[0, :_HALF], p[1, :_HALF]], axis=0)[:_N]
    rsflat = jnp.concatenate(
        [p[0, _FROWS:_FROWS + 80].reshape(-1)[:2 * _HALF],
         p[1, _FROWS:_FROWS + 80].reshape(-1)[:2 * _HALF]])
    out0 = feats_acc[:, 0:64] / rsflat[0:2 * _N:2, None]
    out1 = feats_acc[:, 64:128] / rsflat[1:2 * _N:2, None]
    return jnp.concatenate([out0, out1], axis=1)


# speedup vs baseline: 3.7408x; 1.0452x over previous
import functools

import jax
import jax.numpy as jnp
from jax import lax
from jax.experimental import pallas as pl
from jax.experimental.pallas import tpu as pltpu
from jax.experimental.pallas import tpu_sc as plsc

_N = 10000
_ROW = 128
_NACC = 10112
_HALF = 5056
_FROWS = 5120
_ACCR = 5248
_RSROWS = 88
_CHUNK = 128
_NCHUNK = 80
_EPT = _CHUNK * _NCHUNK
_EPAD = 32 * _EPT
_NCT = _EPAD // _CHUNK
_ZROWS = _ACCR // 16


def _dense_body(x_ref, w1_ref, b1_ref, w2_ref, b2_ref, wa_ref, f_ref, a_ref, m_ref):
    h = jnp.maximum(
        jnp.dot(x_ref[...], w1_ref[...], preferred_element_type=jnp.float32)
        + b1_ref[...], 0.0)
    f = jnp.dot(h, w2_ref[...], preferred_element_type=jnp.float32) + b2_ref[...]
    f_ref[...] = f
    a = jnp.dot(f, wa_ref[...], preferred_element_type=jnp.float32)
    a_ref[...] = a
    m_ref[...] = jnp.broadcast_to(jnp.max(a, axis=0, keepdims=True), (8, 128))[None]


def _dense(x, w1c, b1c, w2bd, b2c, wad):
    return pl.pallas_call(
        _dense_body,
        grid=(10,),
        in_specs=[
            pl.BlockSpec((1000, 128), lambda i: (i, 0)),
            pl.BlockSpec((128, 256), lambda i: (0, 0)),
            pl.BlockSpec((1, 256), lambda i: (0, 0)),
            pl.BlockSpec((256, 128), lambda i: (0, 0)),
            pl.BlockSpec((1, 128), lambda i: (0, 0)),
            pl.BlockSpec((128, 128), lambda i: (0, 0)),
        ],
        out_specs=[
            pl.BlockSpec((1000, 128), lambda i: (i, 0)),
            pl.BlockSpec((1000, 128), lambda i: (i, 0)),
            pl.BlockSpec((1, 8, 128), lambda i: (i, 0, 0)),
        ],
        out_shape=[
            jax.ShapeDtypeStruct((_N, 128), jnp.float32),
            jax.ShapeDtypeStruct((_N, 128), jnp.float32),
            jax.ShapeDtypeStruct((10, 8, 128), jnp.float32),
        ],
    )(x, w1c, b1c, w2bd, b2c, wad)


def _sc_edge_body(htab, g0h, g1h, sdp, elemp, params,
                  fout,
                  se0_v, se1_v, el0_v, el1_v, src0_v, src1_v, dst0_v, dst1_v,
                  e00_v, e01_v, e10_v, e11_v, prm_v,
                  rows0_v, rows1_v, zbuf_v, g0_v, g1_v, rs_v, idxa_v, acc_sh,
                  sem0, sem1):
    cid = lax.axis_index("c")
    sid = lax.axis_index("s")
    wid = sid * 2 + cid
    se_vs = (se0_v, se1_v)
    el_vs = (el0_v, el1_v)
    src_vs = (src0_v, src1_v)
    dst_vs = (dst0_v, dst1_v)
    e0_vs = (e00_v, e01_v)
    e1_vs = (e10_v, e11_v)
    rows_vs = (rows0_v, rows1_v)
    sems = (sem0, sem1)

    zero16 = jnp.zeros((16,), jnp.float32)
    lane = lax.iota(jnp.int32, 16)
    r0 = sid * _ZROWS

    def _zrow(j, carry):
        for k in range(_ROW // 16):
            zbuf_v[j, pl.ds(k * 16, 16)] = zero16
        return carry

    lax.fori_loop(0, _CHUNK, _zrow, 0)

    def _zero_acc_slice():
        for off, size in ((0, 128), (128, 128), (256, 72)):
            pltpu.sync_copy(zbuf_v.at[pl.ds(0, size)], acc_sh.at[pl.ds(r0 + off, size)])

    def _zero_rs():
        def _zrs(j, carry):
            for k in range(8):
                rs_v[j, pl.ds(k * 16, 16)] = zero16
            return carry

        lax.fori_loop(0, _RSROWS, _zrs, 0)

    _zero_acc_slice()
    _zero_rs()

    for gq in range(5):
        idxa_v[pl.ds(gq * 16, 16)] = _FROWS + gq * 16 + lane

    pltpu.sync_copy(g0h, g0_v)
    pltpu.sync_copy(g1h, g1_v)
    pltpu.sync_copy(params, prm_v)
    pv = prm_v[...]
    c0 = jnp.full((16,), pv[0])
    c1 = jnp.full((16,), pv[1])

    plsc.subcore_barrier()

    cbase = wid * _NCHUNK

    def _load(c, p, nbase):
        base = (cbase + c) * _CHUNK
        pltpu.sync_copy(sdp.at[pl.ds(base, _CHUNK)], se_vs[p])
        pltpu.sync_copy(elemp.at[pl.ds(base, _CHUNK)], el_vs[p])
        for gq in range(_CHUNK // 16):
            v = se_vs[p][pl.ds(gq * 16, 16)]
            sl = (v >> 16) - nbase
            ok = (sl >= 0) & (sl < _HALF)
            src_vs[p][pl.ds(gq * 16, 16)] = jnp.where(ok, sl, _HALF)
            dst_vs[p][pl.ds(gq * 16, 16)] = v & 0xFFFF
            ev = el_vs[p][pl.ds(gq * 16, 16)]
            e0_vs[p][pl.ds(gq * 16, 16)] = jnp.exp(c0 * ev)
            e1_vs[p][pl.ds(gq * 16, 16)] = jnp.exp(c1 * ev)
        pltpu.async_copy(htab.at[dst_vs[p]], rows_vs[p], sems[p])

    def _proc(p):
        pltpu.make_async_copy(htab.at[dst_vs[p]], rows_vs[p], sems[p]).wait()
        rows_v = rows_vs[p]

        def _grp(gq, c2):
            e0g = e0_vs[p][pl.ds(gq * 16, 16)]
            e1g = e1_vs[p][pl.ds(gq * 16, 16)]
            sv = src_vs[p][pl.ds(gq * 16, 16)]
            dv = dst_vs[p][pl.ds(gq * 16, 16)]
            for l in range(16):
                j = gq * 16 + l
                d = dv[l]
                s = sv[l]
                w0s = g0_v[pl.ds(d, 16)][0] * e0g[l]
                w1s = g1_v[pl.ds(d, 16)][0] * e1g[l]
                w0 = jnp.full((16,), w0s)
                w1 = jnp.full((16,), w1s)
                for k in range(_ROW // 16):
                    scale = w0 if k < 4 else w1
                    rows_v[j, pl.ds(k * 16, 16)] = rows_v[j, pl.ds(k * 16, 16)] * scale
                p2 = 2 * s
                r = p2 // 128
                cb = p2 % 128
                cb2 = jnp.minimum(cb, 112)
                rel = jnp.full((16,), cb - cb2)
                upd = (jnp.where(lane == rel, w0, 0.0)
                       + jnp.where(lane == rel + 1, w1, 0.0))
                rs_v[r, pl.ds(cb2, 16)] = rs_v[r, pl.ds(cb2, 16)] + upd
            return c2

        lax.fori_loop(0, _CHUNK // 16, _grp, 0)
        pltpu.sync_copy(rows_v, acc_sh.at[src_vs[p]], add=True)

    for ph in range(2):
        nbase = ph * _HALF

        _load(0, 0, nbase)

        def _pair(pr, carry):
            c = 2 * pr
            _load(c + 1, 1, nbase)
            _proc(0)
            _load(c + 2, 0, nbase)
            _proc(1)
            return carry

        lax.fori_loop(0, (_NCHUNK - 2) // 2, _pair, 0)
        _load(_NCHUNK - 1, 1, nbase)
        _proc(0)
        _proc(1)

        pltpu.sync_copy(rs_v.at[pl.ds(0, 80)], acc_sh.at[idxa_v], add=True)
        plsc.subcore_barrier()
        pltpu.sync_copy(acc_sh.at[pl.ds(r0, _ZROWS)], fout.at[cid, ph, pl.ds(r0, _ZROWS)])
        if ph == 0:
            _zero_acc_slice()
            _zero_rs()
            plsc.subcore_barrier()


_sc_edge = pl.kernel(
    _sc_edge_body,
    out_type=jax.ShapeDtypeStruct((2, 2, _ACCR, _ROW), jnp.float32),
    mesh=plsc.VectorSubcoreMesh(core_axis_name="c", subcore_axis_name="s"),
    scratch_types=[
        pltpu.VMEM((_CHUNK,), jnp.int32),
        pltpu.VMEM((_CHUNK,), jnp.int32),
        pltpu.VMEM((_CHUNK,), jnp.float32),
        pltpu.VMEM((_CHUNK,), jnp.float32),
        pltpu.VMEM((_CHUNK,), jnp.int32),
        pltpu.VMEM((_CHUNK,), jnp.int32),
        pltpu.VMEM((_CHUNK,), jnp.int32),
        pltpu.VMEM((_CHUNK,), jnp.int32),
        pltpu.VMEM((_CHUNK,), jnp.float32),
        pltpu.VMEM((_CHUNK,), jnp.float32),
        pltpu.VMEM((_CHUNK,), jnp.float32),
        pltpu.VMEM((_CHUNK,), jnp.float32),
        pltpu.VMEM((16,), jnp.float32),
        pltpu.VMEM((_CHUNK, _ROW), jnp.float32),
        pltpu.VMEM((_CHUNK, _ROW), jnp.float32),
        pltpu.VMEM((_CHUNK, _ROW), jnp.float32),
        pltpu.VMEM((_NACC + 16,), jnp.float32),
        pltpu.VMEM((_NACC + 16,), jnp.float32),
        pltpu.VMEM((_RSROWS, 128), jnp.float32),
        pltpu.VMEM((80,), jnp.int32),
        pltpu.VMEM_SHARED((_ACCR, _ROW), jnp.float32),
        pltpu.SemaphoreType.DMA,
        pltpu.SemaphoreType.DMA,
    ],
)


def kernel(x, elem, W1, b1, W2, b2, Wa, ba, idx):
    f32 = jnp.float32
    w1c = jnp.concatenate([W1[0], W1[1]], axis=1)
    b1c = jnp.concatenate([b1[0], b1[1]])[None, :]
    w2bd = jnp.zeros((256, 128), f32)
    w2bd = w2bd.at[0:128, 0:64].set(W2[0]).at[128:256, 64:128].set(W2[1])
    b2c = jnp.concatenate([b2[0], b2[1]])[None, :]
    wad = jnp.zeros((128, 128), f32)
    wad = wad.at[0:64, 0].set(Wa[0, 64:128, 0]).at[64:128, 1].set(Wa[1, 64:128, 0])

    feats, araw, amax_blk = _dense(x, w1c, b1c, w2bd, b2c, wad)
    a_max = jnp.max(amax_blk[:, 0, :2], axis=0)
    g = jnp.exp(araw[:, :2] - a_max[None, :])

    htab = jnp.zeros((_NACC, _ROW), f32)
    htab = htab.at[:_N, :].set(feats)
    g0h = jnp.zeros((_NACC + 16,), f32).at[:_N].set(g[:, 0])
    g1h = jnp.zeros((_NACC + 16,), f32).at[:_N].set(g[:, 1])

    params = jnp.zeros((16,), f32).at[0].set(Wa[0, 128, 0]).at[1].set(Wa[1, 128, 0])

    npad = _EPAD - idx.shape[1]
    srcp = jnp.concatenate([idx[0], jnp.full((npad,), _N, jnp.int32)])
    dstp = jnp.concatenate([idx[1], jnp.full((npad,), _N, jnp.int32)])
    sdp = (srcp << 16) | dstp
    elemp = jnp.concatenate([elem, jnp.zeros((npad,), f32)])

    fpart = _sc_edge(htab, g0h, g1h, sdp, elemp, params)
    p = fpart[0] + fpart[1]
    feats_acc = jnp.concatenate([p[0, :_HALF], p[1, :_HALF]], axis=0)[:_N]
    rsflat = jnp.concatenate(
        [p[0, _FROWS:_FROWS + 80].reshape(-1)[:2 * _HALF],
         p[1, _FROWS:_FROWS + 80].reshape(-1)[:2 * _HALF]])
    out0 = feats_acc[:, 0:64] / rsflat[0:2 * _N:2, None]
    out1 = feats_acc[:, 64:128] / rsflat[1:2 * _N:2, None]
    return jnp.concatenate([out0, out1], axis=1)


# speedup vs baseline: 3.7477x; 1.0018x over previous
import functools

import jax
import jax.numpy as jnp
from jax import lax
from jax.experimental import pallas as pl
from jax.experimental.pallas import tpu as pltpu
from jax.experimental.pallas import tpu_sc as plsc

_N = 10000
_ROW = 128
_NACC = 10112
_HALF = 5056
_FROWS = 5120
_ACCR = 5248
_RSROWS = 88
_CHUNK = 128
_NCHUNK = 80
_EPT = _CHUNK * _NCHUNK
_EPAD = 32 * _EPT
_NCT = _EPAD // _CHUNK
_ZROWS = _ACCR // 16


def _dense_body(x_ref, w1_ref, b1_ref, w2_ref, b2_ref, wa_ref, f_ref, a_ref, m_ref):
    h = jnp.maximum(
        jnp.dot(x_ref[...], w1_ref[...], preferred_element_type=jnp.float32)
        + b1_ref[...], 0.0)
    f = jnp.dot(h, w2_ref[...], preferred_element_type=jnp.float32) + b2_ref[...]
    f_ref[...] = f
    a = jnp.dot(f, wa_ref[...], preferred_element_type=jnp.float32)
    a_ref[...] = a
    m_ref[...] = jnp.broadcast_to(jnp.max(a, axis=0, keepdims=True), (8, 128))[None]


def _dense(x, w1c, b1c, w2bd, b2c, wad):
    return pl.pallas_call(
        _dense_body,
        grid=(10,),
        in_specs=[
            pl.BlockSpec((1000, 128), lambda i: (i, 0)),
            pl.BlockSpec((128, 256), lambda i: (0, 0)),
            pl.BlockSpec((1, 256), lambda i: (0, 0)),
            pl.BlockSpec((256, 128), lambda i: (0, 0)),
            pl.BlockSpec((1, 128), lambda i: (0, 0)),
            pl.BlockSpec((128, 128), lambda i: (0, 0)),
        ],
        out_specs=[
            pl.BlockSpec((1000, 128), lambda i: (i, 0)),
            pl.BlockSpec((1000, 128), lambda i: (i, 0)),
            pl.BlockSpec((1, 8, 128), lambda i: (i, 0, 0)),
        ],
        out_shape=[
            jax.ShapeDtypeStruct((_N, 128), jnp.float32),
            jax.ShapeDtypeStruct((_N, 128), jnp.float32),
            jax.ShapeDtypeStruct((10, 8, 128), jnp.float32),
        ],
    )(x, w1c, b1c, w2bd, b2c, wad)


def _sc_edge_body(htab, g0h, g1h, sdp, elemp, params,
                  fout,
                  se0_v, se1_v, el0_v, el1_v, src0_v, src1_v, dst0_v, dst1_v,
                  e00_v, e01_v, e10_v, e11_v, prm_v,
                  rows0_v, rows1_v, zbuf_v, g0_v, g1_v, rs_v, idxa_v, acc_sh,
                  sem0, sem1):
    cid = lax.axis_index("c")
    sid = lax.axis_index("s")
    wid = sid * 2 + cid
    se_vs = (se0_v, se1_v)
    el_vs = (el0_v, el1_v)
    src_vs = (src0_v, src1_v)
    dst_vs = (dst0_v, dst1_v)
    e0_vs = (e00_v, e01_v)
    e1_vs = (e10_v, e11_v)
    rows_vs = (rows0_v, rows1_v)
    sems = (sem0, sem1)

    zero16 = jnp.zeros((16,), jnp.float32)
    lane = lax.iota(jnp.int32, 16)
    r0 = sid * _ZROWS

    def _zrow(j, carry):
        for k in range(_ROW // 16):
            zbuf_v[j, pl.ds(k * 16, 16)] = zero16
        return carry

    lax.fori_loop(0, _CHUNK, _zrow, 0)

    def _zero_acc_slice():
        for off, size in ((0, 128), (128, 128), (256, 72)):
            pltpu.sync_copy(zbuf_v.at[pl.ds(0, size)], acc_sh.at[pl.ds(r0 + off, size)])

    def _zero_rs():
        def _zrs(j, carry):
            for k in range(8):
                rs_v[j, pl.ds(k * 16, 16)] = zero16
            return carry

        lax.fori_loop(0, _RSROWS, _zrs, 0)

    _zero_acc_slice()
    _zero_rs()

    for gq in range(5):
        idxa_v[pl.ds(gq * 16, 16)] = _FROWS + gq * 16 + lane

    pltpu.sync_copy(g0h, g0_v)
    pltpu.sync_copy(g1h, g1_v)
    pltpu.sync_copy(params, prm_v)
    pv = prm_v[...]
    c0 = jnp.full((16,), pv[0])
    c1 = jnp.full((16,), pv[1])

    plsc.subcore_barrier()

    cbase = wid * _NCHUNK

    def _load(c, p, nbase):
        base = (cbase + c) * _CHUNK
        pltpu.sync_copy(sdp.at[pl.ds(base, _CHUNK)], se_vs[p])
        pltpu.sync_copy(elemp.at[pl.ds(base, _CHUNK)], el_vs[p])
        for gq in range(_CHUNK // 16):
            v = se_vs[p][pl.ds(gq * 16, 16)]
            sl = (v >> 16) - nbase
            ok = (sl >= 0) & (sl < _HALF)
            dmy = _HALF + (gq * 16) % 64 + lane
            src_vs[p][pl.ds(gq * 16, 16)] = jnp.where(ok, sl, dmy)
            dst_vs[p][pl.ds(gq * 16, 16)] = v & 0xFFFF
            ev = el_vs[p][pl.ds(gq * 16, 16)]
            e0_vs[p][pl.ds(gq * 16, 16)] = jnp.exp(c0 * ev)
            e1_vs[p][pl.ds(gq * 16, 16)] = jnp.exp(c1 * ev)
        pltpu.async_copy(htab.at[dst_vs[p]], rows_vs[p], sems[p])

    def _proc(p):
        pltpu.make_async_copy(htab.at[dst_vs[p]], rows_vs[p], sems[p]).wait()
        rows_v = rows_vs[p]

        def _grp(gq, c2):
            e0g = e0_vs[p][pl.ds(gq * 16, 16)]
            e1g = e1_vs[p][pl.ds(gq * 16, 16)]
            sv = src_vs[p][pl.ds(gq * 16, 16)]
            dv = dst_vs[p][pl.ds(gq * 16, 16)]
            for l in range(16):
                j = gq * 16 + l
                d = dv[l]
                s = sv[l]
                w0s = g0_v[pl.ds(d, 16)][0] * e0g[l]
                w1s = g1_v[pl.ds(d, 16)][0] * e1g[l]
                w0 = jnp.full((16,), w0s)
                w1 = jnp.full((16,), w1s)
                for k in range(_ROW // 16):
                    scale = w0 if k < 4 else w1
                    rows_v[j, pl.ds(k * 16, 16)] = rows_v[j, pl.ds(k * 16, 16)] * scale
                p2 = 2 * s
                r = p2 // 128
                cb = p2 % 128
                cb2 = jnp.minimum(cb, 112)
                rel = jnp.full((16,), cb - cb2)
                upd = (jnp.where(lane == rel, w0, 0.0)
                       + jnp.where(lane == rel + 1, w1, 0.0))
                rs_v[r, pl.ds(cb2, 16)] = rs_v[r, pl.ds(cb2, 16)] + upd
            return c2

        lax.fori_loop(0, _CHUNK // 16, _grp, 0)
        pltpu.sync_copy(rows_v, acc_sh.at[src_vs[p]], add=True)

    for ph in range(2):
        nbase = ph * _HALF

        _load(0, 0, nbase)

        def _pair(pr, carry):
            c = 2 * pr
            _load(c + 1, 1, nbase)
            _proc(0)
            _load(c + 2, 0, nbase)
            _proc(1)
            return carry

        lax.fori_loop(0, (_NCHUNK - 2) // 2, _pair, 0)
        _load(_NCHUNK - 1, 1, nbase)
        _proc(0)
        _proc(1)

        pltpu.sync_copy(rs_v.at[pl.ds(0, 80)], acc_sh.at[idxa_v], add=True)
        plsc.subcore_barrier()
        pltpu.sync_copy(acc_sh.at[pl.ds(r0, _ZROWS)], fout.at[cid, ph, pl.ds(r0, _ZROWS)])
        if ph == 0:
            _zero_acc_slice()
            _zero_rs()
            plsc.subcore_barrier()


_sc_edge = pl.kernel(
    _sc_edge_body,
    out_type=jax.ShapeDtypeStruct((2, 2, _ACCR, _ROW), jnp.float32),
    mesh=plsc.VectorSubcoreMesh(core_axis_name="c", subcore_axis_name="s"),
    scratch_types=[
        pltpu.VMEM((_CHUNK,), jnp.int32),
        pltpu.VMEM((_CHUNK,), jnp.int32),
        pltpu.VMEM((_CHUNK,), jnp.float32),
        pltpu.VMEM((_CHUNK,), jnp.float32),
        pltpu.VMEM((_CHUNK,), jnp.int32),
        pltpu.VMEM((_CHUNK,), jnp.int32),
        pltpu.VMEM((_CHUNK,), jnp.int32),
        pltpu.VMEM((_CHUNK,), jnp.int32),
        pltpu.VMEM((_CHUNK,), jnp.float32),
        pltpu.VMEM((_CHUNK,), jnp.float32),
        pltpu.VMEM((_CHUNK,), jnp.float32),
        pltpu.VMEM((_CHUNK,), jnp.float32),
        pltpu.VMEM((16,), jnp.float32),
        pltpu.VMEM((_CHUNK, _ROW), jnp.float32),
        pltpu.VMEM((_CHUNK, _ROW), jnp.float32),
        pltpu.VMEM((_CHUNK, _ROW), jnp.float32),
        pltpu.VMEM((_NACC + 16,), jnp.float32),
        pltpu.VMEM((_NACC + 16,), jnp.float32),
        pltpu.VMEM((_RSROWS, 128), jnp.float32),
        pltpu.VMEM((80,), jnp.int32),
        pltpu.VMEM_SHARED((_ACCR, _ROW), jnp.float32),
        pltpu.SemaphoreType.DMA,
        pltpu.SemaphoreType.DMA,
    ],
)


def kernel(x, elem, W1, b1, W2, b2, Wa, ba, idx):
    f32 = jnp.float32
    w1c = jnp.concatenate([W1[0], W1[1]], axis=1)
    b1c = jnp.concatenate([b1[0], b1[1]])[None, :]
    w2bd = jnp.zeros((256, 128), f32)
    w2bd = w2bd.at[0:128, 0:64].set(W2[0]).at[128:256, 64:128].set(W2[1])
    b2c = jnp.concatenate([b2[0], b2[1]])[None, :]
    wad = jnp.zeros((128, 128), f32)
    wad = wad.at[0:64, 0].set(Wa[0, 64:128, 0]).at[64:128, 1].set(Wa[1, 64:128, 0])

    feats, araw, amax_blk = _dense(x, w1c, b1c, w2bd, b2c, wad)
    a_max = jnp.max(amax_blk[:, 0, :2], axis=0)
    g = jnp.exp(araw[:, :2] - a_max[None, :])

    htab = jnp.zeros((_NACC, _ROW), f32)
    htab = htab.at[:_N, :].set(feats)
    g0h = jnp.zeros((_NACC + 16,), f32).at[:_N].set(g[:, 0])
    g1h = jnp.zeros((_NACC + 16,), f32).at[:_N].set(g[:, 1])

    params = jnp.zeros((16,), f32).at[0].set(Wa[0, 128, 0]).at[1].set(Wa[1, 128, 0])

    npad = _EPAD - idx.shape[1]
    srcp = jnp.concatenate([idx[0], jnp.full((npad,), _N, jnp.int32)])
    dstp = jnp.concatenate([idx[1], jnp.full((npad,), _N, jnp.int32)])
    sdp = (srcp << 16) | dstp
    elemp = jnp.concatenate([elem, jnp.zeros((npad,), f32)])

    fpart = _sc_edge(htab, g0h, g1h, sdp, elemp, params)
    p = fpart[0] + fpart[1]
    feats_acc = jnp.concatenate([p[0, :_HALF], p[1, :_HALF]], axis=0)[:_N]
    rsflat = jnp.concatenate(
        [p[0, _FROWS:_FROWS + 80].reshape(-1)[:2 * _HALF],
         p[1, _FROWS:_FROWS + 80].reshape(-1)[:2 * _HALF]])
    out0 = feats_acc[:, 0:64] / rsflat[0:2 * _N:2, None]
    out1 = feats_acc[:, 64:128] / rsflat[1:2 * _N:2, None]
    return jnp.concatenate([out0, out1], axis=1)


# speedup vs baseline: 3.7607x; 1.0035x over previous
import functools

import jax
import jax.numpy as jnp
from jax import lax
from jax.experimental import pallas as pl
from jax.experimental.pallas import tpu as pltpu
from jax.experimental.pallas import tpu_sc as plsc

_N = 10000
_ROW = 128
_NACC = 10112
_HALF = 5056
_FROWS = 5120
_ACCR = 5248
_RSFLAT = 10256
_CHUNK = 128
_NCHUNK = 80
_EPT = _CHUNK * _NCHUNK
_EPAD = 32 * _EPT
_NCT = _EPAD // _CHUNK
_ZROWS = _ACCR // 16


def _dense_body(x_ref, w1_ref, b1_ref, w2_ref, b2_ref, wa_ref, f_ref, a_ref, m_ref):
    h = jnp.maximum(
        jnp.dot(x_ref[...], w1_ref[...], preferred_element_type=jnp.float32)
        + b1_ref[...], 0.0)
    f = jnp.dot(h, w2_ref[...], preferred_element_type=jnp.float32) + b2_ref[...]
    f_ref[...] = f
    a = jnp.dot(f, wa_ref[...], preferred_element_type=jnp.float32)
    a_ref[...] = a
    m_ref[...] = jnp.broadcast_to(jnp.max(a, axis=0, keepdims=True), (8, 128))[None]


def _dense(x, w1c, b1c, w2bd, b2c, wad):
    return pl.pallas_call(
        _dense_body,
        grid=(10,),
        in_specs=[
            pl.BlockSpec((1000, 128), lambda i: (i, 0)),
            pl.BlockSpec((128, 256), lambda i: (0, 0)),
            pl.BlockSpec((1, 256), lambda i: (0, 0)),
            pl.BlockSpec((256, 128), lambda i: (0, 0)),
            pl.BlockSpec((1, 128), lambda i: (0, 0)),
            pl.BlockSpec((128, 128), lambda i: (0, 0)),
        ],
        out_specs=[
            pl.BlockSpec((1000, 128), lambda i: (i, 0)),
            pl.BlockSpec((1000, 128), lambda i: (i, 0)),
            pl.BlockSpec((1, 8, 128), lambda i: (i, 0, 0)),
        ],
        out_shape=[
            jax.ShapeDtypeStruct((_N, 128), jnp.float32),
            jax.ShapeDtypeStruct((_N, 128), jnp.float32),
            jax.ShapeDtypeStruct((10, 8, 128), jnp.float32),
        ],
    )(x, w1c, b1c, w2bd, b2c, wad)


def _sc_edge_body(htab, g0h, g1h, sdp, elemp, params,
                  fout,
                  se0_v, se1_v, el0_v, el1_v, src0_v, src1_v, dst0_v, dst1_v,
                  e00_v, e01_v, e10_v, e11_v, prm_v,
                  rows0_v, rows1_v, g0_v, g1_v,
                  rsa_v, rsb_v, idxa_v, acc_sh,
                  sem0, sem1):
    cid = lax.axis_index("c")
    sid = lax.axis_index("s")
    wid = sid * 2 + cid
    se_vs = (se0_v, se1_v)
    el_vs = (el0_v, el1_v)
    src_vs = (src0_v, src1_v)
    dst_vs = (dst0_v, dst1_v)
    e0_vs = (e00_v, e01_v)
    e1_vs = (e10_v, e11_v)
    rows_vs = (rows0_v, rows1_v)
    rs_bufs = (rsa_v, rsb_v)
    sems = (sem0, sem1)

    zero16 = jnp.zeros((16,), jnp.float32)
    lane = lax.iota(jnp.int32, 16)
    r0 = sid * _ZROWS

    def _zero_rows0():
        def _zrow(j, carry):
            for k in range(_ROW // 16):
                rows0_v[j, pl.ds(k * 16, 16)] = zero16
            return carry

        lax.fori_loop(0, _CHUNK, _zrow, 0)

    _zero_rows0()

    def _zero_acc_slice():
        for off, size in ((0, 128), (128, 128), (256, 72)):
            pltpu.sync_copy(rows0_v.at[pl.ds(0, size)], acc_sh.at[pl.ds(r0 + off, size)])

    def _zero_rs():
        def _zrs(j, carry):
            for b in range(2):
                rs_bufs[b][pl.ds(j * 16, 16)] = zero16
            return carry

        lax.fori_loop(0, _RSFLAT // 16, _zrs, 0)

    _zero_acc_slice()
    _zero_rs()

    for gq in range(5):
        idxa_v[pl.ds(gq * 16, 16)] = _FROWS + gq * 16 + lane

    pltpu.sync_copy(g0h, g0_v)
    pltpu.sync_copy(g1h, g1_v)
    pltpu.sync_copy(params, prm_v)
    pv = prm_v[...]
    c0 = jnp.full((16,), pv[0])
    c1 = jnp.full((16,), pv[1])
    m0f = jnp.where(lane == 0, 1.0, 0.0).astype(jnp.float32)
    m1f = jnp.where(lane == 1, 1.0, 0.0).astype(jnp.float32)

    plsc.subcore_barrier()

    cbase = wid * _NCHUNK

    def _load(c, p, nbase):
        base = (cbase + c) * _CHUNK
        pltpu.sync_copy(sdp.at[pl.ds(base, _CHUNK)], se_vs[p])
        pltpu.sync_copy(elemp.at[pl.ds(base, _CHUNK)], el_vs[p])
        for gq in range(_CHUNK // 16):
            v = se_vs[p][pl.ds(gq * 16, 16)]
            sl = (v >> 16) - nbase
            ok = (sl >= 0) & (sl < _HALF)
            dmy = _HALF + (gq * 16) % 64 + lane
            src_vs[p][pl.ds(gq * 16, 16)] = jnp.where(ok, sl, dmy)
            dst_vs[p][pl.ds(gq * 16, 16)] = v & 0xFFFF
            ev = el_vs[p][pl.ds(gq * 16, 16)]
            e0_vs[p][pl.ds(gq * 16, 16)] = jnp.exp(c0 * ev)
            e1_vs[p][pl.ds(gq * 16, 16)] = jnp.exp(c1 * ev)
        pltpu.async_copy(htab.at[dst_vs[p]], rows_vs[p], sems[p])

    def _proc(p):
        pltpu.make_async_copy(htab.at[dst_vs[p]], rows_vs[p], sems[p]).wait()
        rows_v = rows_vs[p]

        def _grp(gq, c2):
            e0g = e0_vs[p][pl.ds(gq * 16, 16)]
            e1g = e1_vs[p][pl.ds(gq * 16, 16)]
            sv = src_vs[p][pl.ds(gq * 16, 16)]
            dv = dst_vs[p][pl.ds(gq * 16, 16)]
            for l in range(16):
                j = gq * 16 + l
                d = dv[l]
                s = sv[l]
                w0s = g0_v[pl.ds(d, 16)][0] * e0g[l]
                w1s = g1_v[pl.ds(d, 16)][0] * e1g[l]
                w0 = jnp.full((16,), w0s)
                w1 = jnp.full((16,), w1s)
                for k in range(_ROW // 16):
                    scale = w0 if k < 4 else w1
                    rows_v[j, pl.ds(k * 16, 16)] = rows_v[j, pl.ds(k * 16, 16)] * scale
                rsb = rs_bufs[l % 2]
                p2 = 2 * s
                upd = w0 * m0f + w1 * m1f
                rsb[pl.ds(p2, 16)] = rsb[pl.ds(p2, 16)] + upd
            return c2

        lax.fori_loop(0, _CHUNK // 16, _grp, 0)
        pltpu.sync_copy(rows_v, acc_sh.at[src_vs[p]], add=True)

    for ph in range(2):
        nbase = ph * _HALF

        _load(0, 0, nbase)

        def _pair(pr, carry):
            c = 2 * pr
            _load(c + 1, 1, nbase)
            _proc(0)
            _load(c + 2, 0, nbase)
            _proc(1)
            return carry

        lax.fori_loop(0, (_NCHUNK - 2) // 2, _pair, 0)
        _load(_NCHUNK - 1, 1, nbase)
        _proc(0)
        _proc(1)

        def _fold(j, carry):
            for k in range(8):
                o = j * 128 + k * 16
                rows0_v[j, pl.ds(k * 16, 16)] = rsa_v[pl.ds(o, 16)] + rsb_v[pl.ds(o, 16)]
            return carry

        lax.fori_loop(0, 80, _fold, 0)
        pltpu.sync_copy(rows0_v.at[pl.ds(0, 80)], acc_sh.at[idxa_v], add=True)
        plsc.subcore_barrier()
        pltpu.sync_copy(acc_sh.at[pl.ds(r0, _ZROWS)], fout.at[cid, ph, pl.ds(r0, _ZROWS)])
        if ph == 0:
            _zero_rows0()
            _zero_acc_slice()
            _zero_rs()
            plsc.subcore_barrier()


_sc_edge = pl.kernel(
    _sc_edge_body,
    out_type=jax.ShapeDtypeStruct((2, 2, _ACCR, _ROW), jnp.float32),
    mesh=plsc.VectorSubcoreMesh(core_axis_name="c", subcore_axis_name="s"),
    scratch_types=[
        pltpu.VMEM((_CHUNK,), jnp.int32),
        pltpu.VMEM((_CHUNK,), jnp.int32),
        pltpu.VMEM((_CHUNK,), jnp.float32),
        pltpu.VMEM((_CHUNK,), jnp.float32),
        pltpu.VMEM((_CHUNK,), jnp.int32),
        pltpu.VMEM((_CHUNK,), jnp.int32),
        pltpu.VMEM((_CHUNK,), jnp.int32),
        pltpu.VMEM((_CHUNK,), jnp.int32),
        pltpu.VMEM((_CHUNK,), jnp.float32),
        pltpu.VMEM((_CHUNK,), jnp.float32),
        pltpu.VMEM((_CHUNK,), jnp.float32),
        pltpu.VMEM((_CHUNK,), jnp.float32),
        pltpu.VMEM((16,), jnp.float32),
        pltpu.VMEM((_CHUNK, _ROW), jnp.float32),
        pltpu.VMEM((_CHUNK, _ROW), jnp.float32),
        pltpu.VMEM((_NACC + 16,), jnp.float32),
        pltpu.VMEM((_NACC + 16,), jnp.float32),
        pltpu.VMEM((_RSFLAT,), jnp.float32),
        pltpu.VMEM((_RSFLAT,), jnp.float32),
        pltpu.VMEM((80,), jnp.int32),
        pltpu.VMEM_SHARED((_ACCR, _ROW), jnp.float32),
        pltpu.SemaphoreType.DMA,
        pltpu.SemaphoreType.DMA,
    ],
)


def kernel(x, elem, W1, b1, W2, b2, Wa, ba, idx):
    f32 = jnp.float32
    w1c = jnp.concatenate([W1[0], W1[1]], axis=1)
    b1c = jnp.concatenate([b1[0], b1[1]])[None, :]
    w2bd = jnp.zeros((256, 128), f32)
    w2bd = w2bd.at[0:128, 0:64].set(W2[0]).at[128:256, 64:128].set(W2[1])
    b2c = jnp.concatenate([b2[0], b2[1]])[None, :]
    wad = jnp.zeros((128, 128), f32)
    wad = wad.at[0:64, 0].set(Wa[0, 64:128, 0]).at[64:128, 1].set(Wa[1, 64:128, 0])

    feats, araw, amax_blk = _dense(x, w1c, b1c, w2bd, b2c, wad)
    a_max = jnp.max(amax_blk[:, 0, :2], axis=0)
    g = jnp.exp(araw[:, :2] - a_max[None, :])

    htab = jnp.zeros((_NACC, _ROW), f32)
    htab = htab.at[:_N, :].set(feats)
    g0h = jnp.zeros((_NACC + 16,), f32).at[:_N].set(g[:, 0])
    g1h = jnp.zeros((_NACC + 16,), f32).at[:_N].set(g[:, 1])

    params = jnp.zeros((16,), f32).at[0].set(Wa[0, 128, 0]).at[1].set(Wa[1, 128, 0])

    npad = _EPAD - idx.shape[1]
    srcp = jnp.concatenate([idx[0], jnp.full((npad,), _N, jnp.int32)])
    dstp = jnp.concatenate([idx[1], jnp.full((npad,), _N, jnp.int32)])
    sdp = (srcp << 16) | dstp
    elemp = jnp.concatenate([elem, jnp.zeros((npad,), f32)])

    fpart = _sc_edge(htab, g0h, g1h, sdp, elemp, params)
    p = fpart[0] + fpart[1]
    feats_acc = jnp.concatenate([p[0, :_HALF], p[1, :_HALF]], axis=0)[:_N]
    rsflat = jnp.concatenate(
        [p[0, _FROWS:_FROWS + 80].reshape(-1)[:2 * _HALF],
         p[1, _FROWS:_FROWS + 80].reshape(-1)[:2 * _HALF]])
    out0 = feats_acc[:, 0:64] / rsflat[0:2 * _N:2, None]
    out1 = feats_acc[:, 64:128] / rsflat[1:2 * _N:2, None]
    return jnp.concatenate([out0, out1], axis=1)
